# baseline (speedup 1.0000x reference)
import jax
import jax.numpy as jnp
from jax import lax
from jax.experimental import pallas as pl
from jax.experimental.pallas import tpu as pltpu
from jax.experimental.pallas import tpu_sc as plsc

_DIM = 64
_BASE = 10000.0

_NC = 2
_NS = 16
_NW = _NC * _NS

_B = 4096
_L = 200
_BG = _B // _NW


def _make_pe():
    pos = jnp.arange(_L, dtype=jnp.float32)[:, None]
    div = jnp.exp(
        jnp.arange(0, _DIM, 2, dtype=jnp.float32) * (-jnp.log(_BASE) / _DIM)
    )
    pe = jnp.zeros((_L, _DIM), dtype=jnp.float32)
    pe = pe.at[:, 0::2].set(jnp.sin(pos * div))
    pe = pe.at[:, 1::2].set(jnp.cos(pos * div))
    return pe


def _sc_body(xt_hbm, table_hbm, pe_hbm, out_hbm, idx_v, pe_v, gbuf, obuf, gsem):
    c = lax.axis_index("c")
    s = lax.axis_index("s")
    wid = s * _NC + c

    pltpu.sync_copy(xt_hbm.at[:, pl.ds(wid * _BG, _BG)], idx_v)
    pltpu.sync_copy(pe_hbm, pe_v)

    row_ids = [lax.iota(jnp.int32, 16) + l * 16 for l in range(_BG // 16)]

    def block(j, _):
        pltpu.async_copy(table_hbm.at[idx_v.at[j]], gbuf, gsem).wait()

        pe_row = [pe_v[j, pl.ds(k * 16, 16)] for k in range(_DIM // 16)]
        for f in range(_DIM):
            pe_f = pe_row[f // 16][f % 16]
            f_ids = lax.full((16,), f, jnp.int32)
            for l in range(_BG // 16):
                col = plsc.load_gather(gbuf, [row_ids[l], f_ids])
                obuf[f // 8, f % 8, pl.ds(l * 16, 16)] = col + pe_f

        pltpu.sync_copy(obuf, out_hbm.at[j, :, wid, :, :])
        return 0

    lax.fori_loop(0, _L, block, 0)


@jax.jit
def kernel(x, table):
    pe = _make_pe()
    xt = x.T.astype(jnp.int32)

    mesh = plsc.VectorSubcoreMesh(core_axis_name="c", subcore_axis_name="s")
    out5 = pl.kernel(
        _sc_body,
        out_type=jax.ShapeDtypeStruct((_L, 8, _NW, 8, _BG), jnp.float32),
        mesh=mesh,
        scratch_types=[
            pltpu.VMEM((_L, _BG), jnp.int32),
            pltpu.VMEM((_L, _DIM), jnp.float32),
            pltpu.VMEM((_BG, _DIM), jnp.float32),
            pltpu.VMEM((8, 8, _BG), jnp.float32),
            pltpu.SemaphoreType.DMA,
        ],
        compiler_params=pltpu.CompilerParams(
            use_tc_tiling_on_sc=False, needs_layout_passes=False
        ),
    )(xt, table, pe)
    return out5.transpose(2, 4, 0, 1, 3).reshape(_B, _L, _DIM)
